# baseline (speedup 1.0000x reference)
import jax
import jax.numpy as jnp
from jax.experimental import pallas as pl
from jax.experimental.pallas import tpu as pltpu

_B = 4096
_DIM = 2048
_HALF = _DIM // 2
_TB = 512


def _v_proj_kernel(x_ref, d_ref, w_ref, b_ref, o_ref, wbf_ref):
    @pl.when(pl.program_id(0) == 0)
    def _():
        wbf_ref[...] = w_ref[...].astype(jnp.bfloat16)

    d = d_ref[...]
    dims = (((1,), (1,)), ((), ()))
    acc = jax.lax.dot_general(
        x_ref[:, 0, :_HALF] + d,
        wbf_ref[:, :_HALF],
        dimension_numbers=dims,
        preferred_element_type=jnp.float32,
    )
    acc += jax.lax.dot_general(
        x_ref[:, 0, _HALF:] + d,
        wbf_ref[:, _HALF:],
        dimension_numbers=dims,
        preferred_element_type=jnp.float32,
    )
    o_ref[...] = acc + b_ref[...]


def kernel(subj, obj, s_o_dot, subj_emb, obj_emb, delta_emb,
           Wq, bq, Wk, bk, Wv, bv, bn_w, bn_b, bn_mean, bn_var):
    bias = bv.reshape(1, _DIM)
    out = pl.pallas_call(
        _v_proj_kernel,
        grid=(_B // _TB,),
        in_specs=[
            pl.BlockSpec((_TB, 1, _DIM), lambda i: (i, 0, 0)),
            pl.BlockSpec((_TB, _HALF), lambda i: (i, 0)),
            pl.BlockSpec((_DIM, _DIM), lambda i: (0, 0)),
            pl.BlockSpec((1, _DIM), lambda i: (0, 0)),
        ],
        out_specs=pl.BlockSpec((_TB, _DIM), lambda i: (i, 0)),
        out_shape=jax.ShapeDtypeStruct((_B, _DIM), jnp.float32),
        scratch_shapes=[pltpu.VMEM((_DIM, _DIM), jnp.bfloat16)],
    )(s_o_dot, delta_emb, Wv, bias)
    return out


# speedup vs baseline: 1.0182x; 1.0182x over previous
import jax
import jax.numpy as jnp
from jax.experimental import pallas as pl

_B = 4096
_DIM = 2048
_HALF = _DIM // 2
_TB = 512


def _v_proj_kernel(x_ref, d_ref, w_ref, b_ref, o_ref):
    d = d_ref[...]
    dims = (((1,), (1,)), ((), ()))
    vl = x_ref[:, 0, :_HALF] + d
    vr = x_ref[:, 0, _HALF:] + d

    def quad(nlo, nhi):
        acc = jax.lax.dot_general(
            vl, w_ref[nlo:nhi, :_HALF],
            dimension_numbers=dims, preferred_element_type=jnp.float32)
        acc += jax.lax.dot_general(
            vr, w_ref[nlo:nhi, _HALF:],
            dimension_numbers=dims, preferred_element_type=jnp.float32)
        return acc + b_ref[:, nlo:nhi]

    o_ref[:, :_HALF] = quad(0, _HALF)
    o_ref[:, _HALF:] = quad(_HALF, _DIM)


def kernel(subj, obj, s_o_dot, subj_emb, obj_emb, delta_emb,
           Wq, bq, Wk, bk, Wv, bv, bn_w, bn_b, bn_mean, bn_var):
    bias = bv.reshape(1, _DIM)
    out = pl.pallas_call(
        _v_proj_kernel,
        grid=(_B // _TB,),
        in_specs=[
            pl.BlockSpec((_TB, 1, _DIM), lambda i: (i, 0, 0)),
            pl.BlockSpec((_TB, _HALF), lambda i: (i, 0)),
            pl.BlockSpec((_DIM, _DIM), lambda i: (0, 0)),
            pl.BlockSpec((1, _DIM), lambda i: (0, 0)),
        ],
        out_specs=pl.BlockSpec((_TB, _DIM), lambda i: (i, 0)),
        out_shape=jax.ShapeDtypeStruct((_B, _DIM), jnp.float32),
    )(s_o_dot, delta_emb, Wv, bias)
    return out
